# baseline (speedup 1.0000x reference)
import functools

import jax
import jax.numpy as jnp
from jax import lax
from jax.experimental import pallas as pl
from jax.experimental.pallas import tpu as pltpu
from jax.experimental.pallas import tpu_sc as plsc

_NC = 2
_NS = 16
_NW = _NC * _NS

_CHUNK = 128
_ZR = 128
_NPAD = 10240
_DH = 64


def _make_prop(n_pad, e, d, npass):
    per_w = e // _NW
    n_chunks = per_w // _CHUNK
    rows_pt = n_pad // _NS
    mesh = plsc.VectorSubcoreMesh(
        core_axis_name="c", subcore_axis_name="s",
        num_cores=_NC, num_subcores=_NS)

    _NB = 4
    n_main = n_chunks // _NB - 1

    @functools.partial(
        pl.kernel,
        out_type=[jax.ShapeDtypeStruct((_NC, n_pad, d), jnp.float32)] * npass,
        mesh=mesh,
        compiler_params=pltpu.CompilerParams(use_tc_tiling_on_sc=(d % 128 == 0)),
        scratch_types=[
            pltpu.VMEM((n_chunks, _CHUNK), jnp.int32),
            pltpu.VMEM((n_chunks, _CHUNK), jnp.int32),
        ] + [pltpu.VMEM((_CHUNK, d), jnp.float32)] * _NB
        + [
            pltpu.VMEM((_ZR, d), jnp.float32),
            pltpu.VMEM_SHARED((n_pad, d), jnp.float32),
        ] + [pltpu.SemaphoreType.DMA] * _NB,
    )
    def prop(*refs):
        xs_tabs = refs[:npass]
        src_hbm, dst_hbm, zeros_hbm = refs[npass:npass + 3]
        outs = refs[npass + 3:2 * npass + 3]
        bufs_and_rest = refs[2 * npass + 5:]
        src_v, dst_v = refs[2 * npass + 3:2 * npass + 5]
        bufs = bufs_and_rest[:_NB]
        zbuf_v = bufs_and_rest[_NB]
        acc = bufs_and_rest[_NB + 1]
        sem_g = bufs_and_rest[_NB + 2:_NB + 2 + _NB]
        cid = lax.axis_index("c")
        sid = lax.axis_index("s")
        wid = cid * _NS + sid
        pltpu.sync_copy(src_hbm.at[wid], src_v)
        pltpu.sync_copy(dst_hbm.at[wid], dst_v)
        pltpu.sync_copy(zeros_hbm, zbuf_v)

        def zero_own():
            for j in range(rows_pt // _ZR):
                pltpu.sync_copy(zbuf_v, acc.at[pl.ds(sid * rows_pt + j * _ZR, _ZR)])

        def run_pass(xs_hbm, out_hbm):
            def start_g(j, b):
                pltpu.async_copy(xs_hbm.at[src_v.at[j]], bufs[b], sem_g[b])

            def finish_g(b):
                pltpu.make_async_copy(xs_hbm.at[src_v.at[0]], bufs[b],
                                      sem_g[b]).wait()

            def scat(j, b):
                pltpu.sync_copy(bufs[b], acc.at[dst_v.at[j]], add=True)

            for b in range(_NB):
                start_g(b, b)

            def body(i, carry):
                for b in range(_NB):
                    finish_g(b)
                    scat(_NB * i + b, b)
                    start_g(_NB * (i + 1) + b, b)
                return carry

            lax.fori_loop(0, n_main, body, 0)
            for c in range(_NB * n_main, n_chunks):
                b = c % _NB
                if c >= _NB * n_main + _NB:
                    start_g(c, b)
                finish_g(b)
                scat(c, b)
            plsc.subcore_barrier()
            pltpu.sync_copy(acc.at[pl.ds(sid * rows_pt, rows_pt)],
                            out_hbm.at[cid, pl.ds(sid * rows_pt, rows_pt)])

        for p in range(npass):
            zero_own()
            plsc.subcore_barrier()
            run_pass(xs_tabs[p], outs[p])

    return prop


def _make_deg(n_pad, e):
    d = 16
    per_w = e // _NW
    n_chunks = per_w // _CHUNK
    rows_pt = n_pad // _NS
    mesh = plsc.VectorSubcoreMesh(
        core_axis_name="c", subcore_axis_name="s",
        num_cores=_NC, num_subcores=_NS)

    @functools.partial(
        pl.kernel,
        out_type=jax.ShapeDtypeStruct((_NC, n_pad, d), jnp.float32),
        mesh=mesh,
        compiler_params=pltpu.CompilerParams(use_tc_tiling_on_sc=False),
        scratch_types=[
            pltpu.VMEM((n_chunks, _CHUNK), jnp.int32),
            pltpu.VMEM((_CHUNK, d), jnp.float32),
            pltpu.VMEM((_ZR, d), jnp.float32),
            pltpu.VMEM_SHARED((n_pad, d), jnp.float32),
        ],
    )
    def deg(ones_hbm, dst_hbm, zeros_hbm, out_hbm, dst_v, ones_v, zbuf_v, acc):
        cid = lax.axis_index("c")
        sid = lax.axis_index("s")
        wid = cid * _NS + sid
        pltpu.sync_copy(dst_hbm.at[wid], dst_v)
        pltpu.sync_copy(ones_hbm, ones_v)
        pltpu.sync_copy(zeros_hbm, zbuf_v)
        for j in range(rows_pt // _ZR):
            pltpu.sync_copy(zbuf_v, acc.at[pl.ds(sid * rows_pt + j * _ZR, _ZR)])
        plsc.subcore_barrier()

        def body(j, carry):
            pltpu.sync_copy(ones_v, acc.at[dst_v.at[j]], add=True)
            return carry

        lax.fori_loop(0, n_chunks, body, 0)
        plsc.subcore_barrier()
        pltpu.sync_copy(acc.at[pl.ds(sid * rows_pt, rows_pt)],
                        out_hbm.at[cid, pl.ds(sid * rows_pt, rows_pt)])

    return deg


_RB = 1024
_EPS = 1e-5


def _in_relu(s):
    m = jnp.mean(s, axis=1, keepdims=True)
    c = s - m
    v = jnp.mean(c * c, axis=1, keepdims=True)
    return jnp.maximum(c * lax.rsqrt(v + _EPS), 0.0)


def _m1_body(x_ref, w1_ref, wr_ref, br_ref, xw1_ref, xwr_ref):
    x = x_ref[...]
    xw1_ref[...] = jnp.dot(x, w1_ref[...], preferred_element_type=jnp.float32)
    xwr_ref[...] = (jnp.dot(x, wr_ref[...], preferred_element_type=jnp.float32)
                    + br_ref[...])


def _s1_body(da_ref, db_ref, xw1_ref, xs1lo_ref, xs1hi_ref,
             dinv_ref, invdeg_ref):
    deg = da_ref[...][:, :1] + db_ref[...][:, :1] + 1.0
    dinv = lax.rsqrt(deg)
    invdeg = 1.0 / deg
    xs1 = dinv * xw1_ref[...]
    xs1lo_ref[...] = xs1[:, :_DH]
    xs1hi_ref[...] = xs1[:, _DH:]
    dinv_ref[...] = dinv
    invdeg_ref[...] = invdeg


def _t1_body(palo_ref, pahi_ref, pblo_ref, pbhi_ref, xw1_ref, dinv_ref,
             invdeg_ref, b1_ref, w2_ref, xw2_ref, xs2lo_ref, xs2hi_ref):
    dinv = dinv_ref[...]
    p = jnp.concatenate([palo_ref[...] + pblo_ref[...],
                         pahi_ref[...] + pbhi_ref[...]], axis=1)
    s = dinv * p + invdeg_ref[...] * xw1_ref[...] + b1_ref[...]
    h1 = _in_relu(s)
    xw2 = jnp.dot(h1, w2_ref[...], preferred_element_type=jnp.float32)
    xs2 = dinv * xw2
    xw2_ref[...] = xw2
    xs2lo_ref[...] = xs2[:, :_DH]
    xs2hi_ref[...] = xs2[:, _DH:]


def _t2_body(palo_ref, pahi_ref, pblo_ref, pbhi_ref, xw2_ref, dinv_ref,
             invdeg_ref, b2_ref, xwr_ref, w3_ref, h_ref, hw3_ref, xs3_ref):
    dinv = dinv_ref[...]
    p = jnp.concatenate([palo_ref[...] + pblo_ref[...],
                         pahi_ref[...] + pbhi_ref[...]], axis=1)
    s = dinv * p + invdeg_ref[...] * xw2_ref[...] + b2_ref[...]
    h = _in_relu(s) + xwr_ref[...]
    hw3 = jnp.dot(h, w3_ref[...], preferred_element_type=jnp.float32)
    h_ref[...] = h
    hw3_ref[...] = hw3
    xs3_ref[...] = dinv * hw3


def _t3_body(pa_ref, pb_ref, hw3_ref, dinv_ref, invdeg_ref, b3_ref, out_ref):
    s = (dinv_ref[...] * (pa_ref[...] + pb_ref[...])
         + invdeg_ref[...] * hw3_ref[...] + b3_ref[...])
    out_ref[...] = jax.nn.sigmoid(s) - 0.5


def _row_spec(d):
    return pl.BlockSpec((_RB, d), lambda i: (i, 0))


def _full_spec(r, c):
    return pl.BlockSpec((r, c), lambda i: (0, 0))


def kernel(x, edge_index, W1, b1, W2, b2, Wr, br, W3, b3):
    n, in_dim = x.shape
    hid = W1.shape[1]
    e = edge_index.shape[1]
    e_pad = -(-e // (_NW * _CHUNK)) * (_NW * _CHUNK)
    per_w = e_pad // _NW
    n_chunks = per_w // _CHUNK
    i32 = jnp.int32
    fill = jnp.full((e_pad - e,), _NPAD - 1, i32)
    src = jnp.concatenate([edge_index[0], fill]).reshape(_NW, n_chunks, _CHUNK)
    dst = jnp.concatenate([edge_index[1], fill]).reshape(_NW, n_chunks, _CHUNK)
    x = jnp.pad(x, ((0, _NPAD - n), (0, 0)))

    prop_h = _make_prop(_NPAD, e_pad, _DH, 2)
    prop_16 = _make_prop(_NPAD, e_pad, 16, 1)
    deg_k = _make_deg(_NPAD, e_pad)
    zeros_h = jnp.zeros((_ZR, _DH), jnp.float32)
    zeros_16 = jnp.zeros((_ZR, 16), jnp.float32)
    ones_rows = jnp.ones((_CHUNK, 16), jnp.float32)

    grid = (_NPAD // _RB,)
    f32 = jnp.float32

    degp = deg_k(ones_rows, dst, zeros_16)
    xw1, xwr = pl.pallas_call(
        _m1_body,
        grid=grid,
        in_specs=[_row_spec(in_dim), _full_spec(in_dim, hid),
                  _full_spec(in_dim, hid), _full_spec(1, hid)],
        out_specs=[_row_spec(hid), _row_spec(hid)],
        out_shape=[jax.ShapeDtypeStruct((_NPAD, hid), f32)] * 2,
    )(x, W1, Wr, br.reshape(1, hid))

    xs1lo, xs1hi, dinv, invdeg = pl.pallas_call(
        _s1_body,
        grid=grid,
        in_specs=[_row_spec(16), _row_spec(16), _row_spec(hid)],
        out_specs=[_row_spec(_DH), _row_spec(_DH), _row_spec(1), _row_spec(1)],
        out_shape=[jax.ShapeDtypeStruct((_NPAD, _DH), f32)] * 2
        + [jax.ShapeDtypeStruct((_NPAD, 1), f32)] * 2,
    )(degp[0], degp[1], xw1)

    p1lo, p1hi = prop_h(xs1lo, xs1hi, src, dst, zeros_h)
    xw2, xs2lo, xs2hi = pl.pallas_call(
        _t1_body,
        grid=grid,
        in_specs=[_row_spec(_DH)] * 4
        + [_row_spec(hid), _row_spec(1), _row_spec(1), _full_spec(1, hid),
           _full_spec(hid, hid)],
        out_specs=[_row_spec(hid), _row_spec(_DH), _row_spec(_DH)],
        out_shape=[jax.ShapeDtypeStruct((_NPAD, hid), f32)]
        + [jax.ShapeDtypeStruct((_NPAD, _DH), f32)] * 2,
    )(p1lo[0], p1hi[0], p1lo[1], p1hi[1], xw1, dinv, invdeg,
      b1.reshape(1, hid), W2)

    w3p = jnp.zeros((hid, 16), f32).at[:, : W3.shape[1]].set(W3)
    b3p = jnp.zeros((1, 16), f32).at[0, : W3.shape[1]].set(b3)
    p2lo, p2hi = prop_h(xs2lo, xs2hi, src, dst, zeros_h)
    h, hw3, xs3 = pl.pallas_call(
        _t2_body,
        grid=grid,
        in_specs=[_row_spec(_DH)] * 4
        + [_row_spec(hid), _row_spec(1), _row_spec(1), _full_spec(1, hid),
           _row_spec(hid), _full_spec(hid, 16)],
        out_specs=[_row_spec(hid), _row_spec(16), _row_spec(16)],
        out_shape=[jax.ShapeDtypeStruct((_NPAD, hid), f32),
                   jax.ShapeDtypeStruct((_NPAD, 16), f32),
                   jax.ShapeDtypeStruct((_NPAD, 16), f32)],
    )(p2lo[0], p2hi[0], p2lo[1], p2hi[1], xw2, dinv, invdeg,
      b2.reshape(1, hid), xwr, w3p)

    (p3,) = prop_16(xs3, src, dst, zeros_16)
    outp = pl.pallas_call(
        _t3_body,
        grid=grid,
        in_specs=[_row_spec(16), _row_spec(16), _row_spec(16),
                  _row_spec(1), _row_spec(1), _full_spec(1, 16)],
        out_specs=_row_spec(16),
        out_shape=jax.ShapeDtypeStruct((_NPAD, 16), f32),
    )(p3[0], p3[1], hw3, dinv, invdeg, b3p)

    return (h[:n], outp[:n, : W3.shape[1]])


# speedup vs baseline: 1.7773x; 1.7773x over previous
import functools

import jax
import jax.numpy as jnp
from jax import lax
from jax.experimental import pallas as pl
from jax.experimental.pallas import tpu as pltpu
from jax.experimental.pallas import tpu_sc as plsc

_NC = 2
_NS = 16
_NW = _NC * _NS

_CHUNK = 80
_ZR = 128
_NPAD = 10240
_DH = 64


def _make_prop(n_pad, e, d, npass):
    per_w = e // _NW
    n_chunks = per_w // _CHUNK
    rows_pt = n_pad // _NS
    mesh = plsc.VectorSubcoreMesh(
        core_axis_name="c", subcore_axis_name="s",
        num_cores=_NC, num_subcores=_NS)

    _NB = 4
    n_main = n_chunks // _NB - 1

    @functools.partial(
        pl.kernel,
        out_type=[jax.ShapeDtypeStruct((_NC, n_pad, d), jnp.float32)] * npass,
        mesh=mesh,
        compiler_params=pltpu.CompilerParams(use_tc_tiling_on_sc=(d % 128 == 0)),
        scratch_types=[
            pltpu.VMEM((n_chunks, _CHUNK), jnp.int32),
            pltpu.VMEM((n_chunks, _CHUNK), jnp.int32),
        ] + [pltpu.VMEM((_CHUNK, d), jnp.float32)] * _NB
        + [
            pltpu.VMEM((_ZR, d), jnp.float32),
            pltpu.VMEM_SHARED((n_pad, d), jnp.float32),
        ] + [pltpu.SemaphoreType.DMA] * _NB,
    )
    def prop(*refs):
        xs_tabs = refs[:npass]
        src_hbm, dst_hbm, zeros_hbm = refs[npass:npass + 3]
        outs = refs[npass + 3:2 * npass + 3]
        bufs_and_rest = refs[2 * npass + 5:]
        src_v, dst_v = refs[2 * npass + 3:2 * npass + 5]
        bufs = bufs_and_rest[:_NB]
        zbuf_v = bufs_and_rest[_NB]
        acc = bufs_and_rest[_NB + 1]
        sem_g = bufs_and_rest[_NB + 2:_NB + 2 + _NB]
        cid = lax.axis_index("c")
        sid = lax.axis_index("s")
        wid = cid * _NS + sid
        pltpu.sync_copy(src_hbm.at[wid], src_v)
        pltpu.sync_copy(dst_hbm.at[wid], dst_v)
        pltpu.sync_copy(zeros_hbm, zbuf_v)

        def zero_own():
            for j in range(rows_pt // _ZR):
                pltpu.sync_copy(zbuf_v, acc.at[pl.ds(sid * rows_pt + j * _ZR, _ZR)])

        def run_pass(xs_hbm, out_hbm):
            def start_g(j, b):
                pltpu.async_copy(xs_hbm.at[src_v.at[j]], bufs[b], sem_g[b])

            def finish_g(b):
                pltpu.make_async_copy(xs_hbm.at[src_v.at[0]], bufs[b],
                                      sem_g[b]).wait()

            def scat(j, b):
                pltpu.sync_copy(bufs[b], acc.at[dst_v.at[j]], add=True)

            for b in range(_NB):
                start_g(b, b)

            def body(i, carry):
                for b in range(_NB):
                    finish_g(b)
                    scat(_NB * i + b, b)
                    start_g(_NB * (i + 1) + b, b)
                return carry

            lax.fori_loop(0, n_main, body, 0)
            for c in range(_NB * n_main, n_chunks):
                b = c % _NB
                if c >= _NB * n_main + _NB:
                    start_g(c, b)
                finish_g(b)
                scat(c, b)
            plsc.subcore_barrier()
            pltpu.sync_copy(acc.at[pl.ds(sid * rows_pt, rows_pt)],
                            out_hbm.at[cid, pl.ds(sid * rows_pt, rows_pt)])

        for p in range(npass):
            zero_own()
            plsc.subcore_barrier()
            run_pass(xs_tabs[p], outs[p])

    return prop


def _make_deg(n_pad, e):
    d = 16
    per_w = e // _NW
    n_chunks = per_w // _CHUNK
    rows_pt = n_pad // _NS
    mesh = plsc.VectorSubcoreMesh(
        core_axis_name="c", subcore_axis_name="s",
        num_cores=_NC, num_subcores=_NS)

    @functools.partial(
        pl.kernel,
        out_type=jax.ShapeDtypeStruct((_NC, n_pad, d), jnp.float32),
        mesh=mesh,
        compiler_params=pltpu.CompilerParams(use_tc_tiling_on_sc=False),
        scratch_types=[
            pltpu.VMEM((n_chunks, _CHUNK), jnp.int32),
            pltpu.VMEM((_CHUNK, d), jnp.float32),
            pltpu.VMEM((_ZR, d), jnp.float32),
            pltpu.VMEM_SHARED((n_pad, d), jnp.float32),
        ],
    )
    def deg(ones_hbm, dst_hbm, zeros_hbm, out_hbm, dst_v, ones_v, zbuf_v, acc):
        cid = lax.axis_index("c")
        sid = lax.axis_index("s")
        wid = cid * _NS + sid
        pltpu.sync_copy(dst_hbm.at[wid], dst_v)
        pltpu.sync_copy(ones_hbm, ones_v)
        pltpu.sync_copy(zeros_hbm, zbuf_v)
        for j in range(rows_pt // _ZR):
            pltpu.sync_copy(zbuf_v, acc.at[pl.ds(sid * rows_pt + j * _ZR, _ZR)])
        plsc.subcore_barrier()

        def body(j, carry):
            pltpu.sync_copy(ones_v, acc.at[dst_v.at[j]], add=True)
            return carry

        lax.fori_loop(0, n_chunks, body, 0)
        plsc.subcore_barrier()
        pltpu.sync_copy(acc.at[pl.ds(sid * rows_pt, rows_pt)],
                        out_hbm.at[cid, pl.ds(sid * rows_pt, rows_pt)])

    return deg


_RB = 1024
_EPS = 1e-5


def _in_relu(s):
    m = jnp.mean(s, axis=1, keepdims=True)
    c = s - m
    v = jnp.mean(c * c, axis=1, keepdims=True)
    return jnp.maximum(c * lax.rsqrt(v + _EPS), 0.0)


def _m1_body(x_ref, w1_ref, wr_ref, br_ref, xw1_ref, xwr_ref):
    x = x_ref[...]
    xw1_ref[...] = jnp.dot(x, w1_ref[...], preferred_element_type=jnp.float32)
    xwr_ref[...] = (jnp.dot(x, wr_ref[...], preferred_element_type=jnp.float32)
                    + br_ref[...])


def _s1_body(da_ref, db_ref, xw1_ref, xs1lo_ref, xs1hi_ref,
             dinv_ref, invdeg_ref):
    deg = da_ref[...][:, :1] + db_ref[...][:, :1] + 1.0
    dinv = lax.rsqrt(deg)
    invdeg = 1.0 / deg
    xs1 = dinv * xw1_ref[...]
    xs1lo_ref[...] = xs1[:, :_DH]
    xs1hi_ref[...] = xs1[:, _DH:]
    dinv_ref[...] = dinv
    invdeg_ref[...] = invdeg


def _t1_body(palo_ref, pahi_ref, pblo_ref, pbhi_ref, xw1_ref, dinv_ref,
             invdeg_ref, b1_ref, w2_ref, xw2_ref, xs2lo_ref, xs2hi_ref):
    dinv = dinv_ref[...]
    p = jnp.concatenate([palo_ref[...] + pblo_ref[...],
                         pahi_ref[...] + pbhi_ref[...]], axis=1)
    s = dinv * p + invdeg_ref[...] * xw1_ref[...] + b1_ref[...]
    h1 = _in_relu(s)
    xw2 = jnp.dot(h1, w2_ref[...], preferred_element_type=jnp.float32)
    xs2 = dinv * xw2
    xw2_ref[...] = xw2
    xs2lo_ref[...] = xs2[:, :_DH]
    xs2hi_ref[...] = xs2[:, _DH:]


def _t2_body(palo_ref, pahi_ref, pblo_ref, pbhi_ref, xw2_ref, dinv_ref,
             invdeg_ref, b2_ref, xwr_ref, w3_ref, h_ref, hw3_ref, xs3_ref):
    dinv = dinv_ref[...]
    p = jnp.concatenate([palo_ref[...] + pblo_ref[...],
                         pahi_ref[...] + pbhi_ref[...]], axis=1)
    s = dinv * p + invdeg_ref[...] * xw2_ref[...] + b2_ref[...]
    h = _in_relu(s) + xwr_ref[...]
    hw3 = jnp.dot(h, w3_ref[...], preferred_element_type=jnp.float32)
    h_ref[...] = h
    hw3_ref[...] = hw3
    xs3_ref[...] = dinv * hw3


def _t3_body(pa_ref, pb_ref, hw3_ref, dinv_ref, invdeg_ref, b3_ref, out_ref):
    s = (dinv_ref[...] * (pa_ref[...] + pb_ref[...])
         + invdeg_ref[...] * hw3_ref[...] + b3_ref[...])
    out_ref[...] = jax.nn.sigmoid(s) - 0.5


def _row_spec(d):
    return pl.BlockSpec((_RB, d), lambda i: (i, 0))


def _full_spec(r, c):
    return pl.BlockSpec((r, c), lambda i: (0, 0))


def kernel(x, edge_index, W1, b1, W2, b2, Wr, br, W3, b3):
    n, in_dim = x.shape
    hid = W1.shape[1]
    e = edge_index.shape[1]
    e_pad = -(-e // (_NW * _CHUNK)) * (_NW * _CHUNK)
    per_w = e_pad // _NW
    n_chunks = per_w // _CHUNK
    i32 = jnp.int32
    fill = jnp.full((e_pad - e,), _NPAD - 1, i32)
    src = jnp.concatenate([edge_index[0], fill]).reshape(_NW, n_chunks, _CHUNK)
    dst = jnp.concatenate([edge_index[1], fill]).reshape(_NW, n_chunks, _CHUNK)
    x = jnp.pad(x, ((0, _NPAD - n), (0, 0)))

    prop_h = _make_prop(_NPAD, e_pad, _DH, 2)
    prop_16 = _make_prop(_NPAD, e_pad, 16, 1)
    deg_k = _make_deg(_NPAD, e_pad)
    zeros_h = jnp.zeros((_ZR, _DH), jnp.float32)
    zeros_16 = jnp.zeros((_ZR, 16), jnp.float32)
    ones_rows = jnp.ones((_CHUNK, 16), jnp.float32)

    grid = (_NPAD // _RB,)
    f32 = jnp.float32

    degp = deg_k(ones_rows, dst, zeros_16)
    xw1, xwr = pl.pallas_call(
        _m1_body,
        grid=grid,
        in_specs=[_row_spec(in_dim), _full_spec(in_dim, hid),
                  _full_spec(in_dim, hid), _full_spec(1, hid)],
        out_specs=[_row_spec(hid), _row_spec(hid)],
        out_shape=[jax.ShapeDtypeStruct((_NPAD, hid), f32)] * 2,
    )(x, W1, Wr, br.reshape(1, hid))

    xs1lo, xs1hi, dinv, invdeg = pl.pallas_call(
        _s1_body,
        grid=grid,
        in_specs=[_row_spec(16), _row_spec(16), _row_spec(hid)],
        out_specs=[_row_spec(_DH), _row_spec(_DH), _row_spec(1), _row_spec(1)],
        out_shape=[jax.ShapeDtypeStruct((_NPAD, _DH), f32)] * 2
        + [jax.ShapeDtypeStruct((_NPAD, 1), f32)] * 2,
    )(degp[0], degp[1], xw1)

    p1lo, p1hi = prop_h(xs1lo, xs1hi, src, dst, zeros_h)
    xw2, xs2lo, xs2hi = pl.pallas_call(
        _t1_body,
        grid=grid,
        in_specs=[_row_spec(_DH)] * 4
        + [_row_spec(hid), _row_spec(1), _row_spec(1), _full_spec(1, hid),
           _full_spec(hid, hid)],
        out_specs=[_row_spec(hid), _row_spec(_DH), _row_spec(_DH)],
        out_shape=[jax.ShapeDtypeStruct((_NPAD, hid), f32)]
        + [jax.ShapeDtypeStruct((_NPAD, _DH), f32)] * 2,
    )(p1lo[0], p1hi[0], p1lo[1], p1hi[1], xw1, dinv, invdeg,
      b1.reshape(1, hid), W2)

    w3p = jnp.zeros((hid, 16), f32).at[:, : W3.shape[1]].set(W3)
    b3p = jnp.zeros((1, 16), f32).at[0, : W3.shape[1]].set(b3)
    p2lo, p2hi = prop_h(xs2lo, xs2hi, src, dst, zeros_h)
    h, hw3, xs3 = pl.pallas_call(
        _t2_body,
        grid=grid,
        in_specs=[_row_spec(_DH)] * 4
        + [_row_spec(hid), _row_spec(1), _row_spec(1), _full_spec(1, hid),
           _row_spec(hid), _full_spec(hid, 16)],
        out_specs=[_row_spec(hid), _row_spec(16), _row_spec(16)],
        out_shape=[jax.ShapeDtypeStruct((_NPAD, hid), f32),
                   jax.ShapeDtypeStruct((_NPAD, 16), f32),
                   jax.ShapeDtypeStruct((_NPAD, 16), f32)],
    )(p2lo[0], p2hi[0], p2lo[1], p2hi[1], xw2, dinv, invdeg,
      b2.reshape(1, hid), xwr, w3p)

    (p3,) = prop_16(xs3, src, dst, zeros_16)
    outp = pl.pallas_call(
        _t3_body,
        grid=grid,
        in_specs=[_row_spec(16), _row_spec(16), _row_spec(16),
                  _row_spec(1), _row_spec(1), _full_spec(1, 16)],
        out_specs=_row_spec(16),
        out_shape=jax.ShapeDtypeStruct((_NPAD, 16), f32),
    )(p3[0], p3[1], hw3, dinv, invdeg, b3p)

    return (h[:n], outp[:n, : W3.shape[1]])


# speedup vs baseline: 1.7993x; 1.0124x over previous
import functools

import jax
import jax.numpy as jnp
from jax import lax
from jax.experimental import pallas as pl
from jax.experimental.pallas import tpu as pltpu
from jax.experimental.pallas import tpu_sc as plsc

_NC = 2
_NS = 16
_NW = _NC * _NS

_CHUNK = 80
_ZR = 128
_NPAD = 10240
_DH = 64


def _make_prop(n_pad, e, d, npass):
    per_w = e // _NW
    n_chunks = per_w // _CHUNK
    rows_pt = n_pad // _NS
    mesh = plsc.VectorSubcoreMesh(
        core_axis_name="c", subcore_axis_name="s",
        num_cores=_NC, num_subcores=_NS)

    _NB = 4
    n_main = n_chunks // _NB - 1

    @functools.partial(
        pl.kernel,
        out_type=[jax.ShapeDtypeStruct((_NC, n_pad, d), jnp.float32)] * npass,
        mesh=mesh,
        compiler_params=pltpu.CompilerParams(use_tc_tiling_on_sc=(d % 128 == 0)),
        scratch_types=[
            pltpu.VMEM((n_chunks, _CHUNK), jnp.int32),
            pltpu.VMEM((n_chunks, _CHUNK), jnp.int32),
        ] + [pltpu.VMEM((_CHUNK, d), jnp.float32)] * _NB
        + [
            pltpu.VMEM((_ZR, d), jnp.float32),
            pltpu.VMEM_SHARED((n_pad, d), jnp.float32),
        ] + [pltpu.SemaphoreType.DMA] * _NB,
    )
    def prop(*refs):
        xs_tabs = refs[:npass]
        src_hbm, dst_hbm, zeros_hbm = refs[npass:npass + 3]
        outs = refs[npass + 3:2 * npass + 3]
        bufs_and_rest = refs[2 * npass + 5:]
        src_v, dst_v = refs[2 * npass + 3:2 * npass + 5]
        bufs = bufs_and_rest[:_NB]
        zbuf_v = bufs_and_rest[_NB]
        acc = bufs_and_rest[_NB + 1]
        sem_g = bufs_and_rest[_NB + 2:_NB + 2 + _NB]
        cid = lax.axis_index("c")
        sid = lax.axis_index("s")
        wid = cid * _NS + sid
        pltpu.sync_copy(src_hbm.at[wid], src_v)
        pltpu.sync_copy(dst_hbm.at[wid], dst_v)
        pltpu.sync_copy(zeros_hbm, zbuf_v)

        def zero_own():
            for j in range(rows_pt // _ZR):
                pltpu.sync_copy(zbuf_v, acc.at[pl.ds(sid * rows_pt + j * _ZR, _ZR)])

        def run_pass(xs_hbm, out_hbm):
            def start_g(j, b):
                pltpu.async_copy(xs_hbm.at[src_v.at[j]], bufs[b], sem_g[b])

            def finish_g(b):
                pltpu.make_async_copy(xs_hbm.at[src_v.at[0]], bufs[b],
                                      sem_g[b]).wait()

            def scat(j, b):
                pltpu.sync_copy(bufs[b], acc.at[dst_v.at[j]], add=True)

            for b in range(_NB):
                start_g(b, b)

            def body(i, carry):
                for b in range(_NB):
                    finish_g(b)
                    scat(_NB * i + b, b)
                    start_g(_NB * (i + 1) + b, b)
                return carry

            lax.fori_loop(0, n_main, body, 0)
            for c in range(_NB * n_main, n_chunks):
                b = c % _NB
                if c >= _NB * n_main + _NB:
                    start_g(c, b)
                finish_g(b)
                scat(c, b)
            plsc.subcore_barrier()
            pltpu.sync_copy(acc.at[pl.ds(sid * rows_pt, rows_pt)],
                            out_hbm.at[cid, pl.ds(sid * rows_pt, rows_pt)])

        for p in range(npass):
            zero_own()
            plsc.subcore_barrier()
            run_pass(xs_tabs[p], outs[p])

    return prop


def _make_deg(n_pad, e):
    d = 16
    per_w = e // _NW
    n_chunks = per_w // _CHUNK
    rows_pt = n_pad // _NS
    mesh = plsc.VectorSubcoreMesh(
        core_axis_name="c", subcore_axis_name="s",
        num_cores=_NC, num_subcores=_NS)

    @functools.partial(
        pl.kernel,
        out_type=jax.ShapeDtypeStruct((_NC, n_pad, d), jnp.float32),
        mesh=mesh,
        compiler_params=pltpu.CompilerParams(use_tc_tiling_on_sc=False),
        scratch_types=[
            pltpu.VMEM((n_chunks, _CHUNK), jnp.int32),
            pltpu.VMEM((_CHUNK, d), jnp.float32),
            pltpu.VMEM((_ZR, d), jnp.float32),
            pltpu.VMEM_SHARED((n_pad, d), jnp.float32),
        ],
    )
    def deg(ones_hbm, dst_hbm, zeros_hbm, out_hbm, dst_v, ones_v, zbuf_v, acc):
        cid = lax.axis_index("c")
        sid = lax.axis_index("s")
        wid = cid * _NS + sid
        pltpu.sync_copy(dst_hbm.at[wid], dst_v)
        pltpu.sync_copy(ones_hbm, ones_v)
        pltpu.sync_copy(zeros_hbm, zbuf_v)
        for j in range(rows_pt // _ZR):
            pltpu.sync_copy(zbuf_v, acc.at[pl.ds(sid * rows_pt + j * _ZR, _ZR)])
        plsc.subcore_barrier()

        def body(j, carry):
            pltpu.sync_copy(ones_v, acc.at[dst_v.at[j]], add=True)
            return carry

        lax.fori_loop(0, n_chunks, body, 0)
        plsc.subcore_barrier()
        pltpu.sync_copy(acc.at[pl.ds(sid * rows_pt, rows_pt)],
                        out_hbm.at[cid, pl.ds(sid * rows_pt, rows_pt)])

    return deg


_RB = 1000
_EPS = 1e-5


def _in_relu(s):
    m = jnp.mean(s, axis=1, keepdims=True)
    c = s - m
    v = jnp.mean(c * c, axis=1, keepdims=True)
    return jnp.maximum(c * lax.rsqrt(v + _EPS), 0.0)


def _m1_body(x_ref, w1_ref, wr_ref, br_ref, xw1_ref, xwr_ref):
    x = x_ref[...]
    xw1_ref[...] = jnp.dot(x, w1_ref[...], preferred_element_type=jnp.float32)
    xwr_ref[...] = (jnp.dot(x, wr_ref[...], preferred_element_type=jnp.float32)
                    + br_ref[...])


def _s1_body(da_ref, db_ref, xw1_ref, xs1lo_ref, xs1hi_ref,
             dinv_ref, invdeg_ref):
    deg = da_ref[...][:, :1] + db_ref[...][:, :1] + 1.0
    dinv = lax.rsqrt(deg)
    invdeg = 1.0 / deg
    xs1 = dinv * xw1_ref[...]
    xs1lo_ref[...] = xs1[:, :_DH]
    xs1hi_ref[...] = xs1[:, _DH:]
    dinv_ref[...] = dinv
    invdeg_ref[...] = invdeg


def _t1_body(palo_ref, pahi_ref, pblo_ref, pbhi_ref, xw1_ref, dinv_ref,
             invdeg_ref, b1_ref, w2_ref, xw2_ref, xs2lo_ref, xs2hi_ref):
    dinv = dinv_ref[...]
    p = jnp.concatenate([palo_ref[...] + pblo_ref[...],
                         pahi_ref[...] + pbhi_ref[...]], axis=1)
    s = dinv * p + invdeg_ref[...] * xw1_ref[...] + b1_ref[...]
    h1 = _in_relu(s)
    xw2 = jnp.dot(h1, w2_ref[...], preferred_element_type=jnp.float32)
    xs2 = dinv * xw2
    xw2_ref[...] = xw2
    xs2lo_ref[...] = xs2[:, :_DH]
    xs2hi_ref[...] = xs2[:, _DH:]


def _t2_body(palo_ref, pahi_ref, pblo_ref, pbhi_ref, xw2_ref, dinv_ref,
             invdeg_ref, b2_ref, xwr_ref, w3_ref, h_ref, hw3_ref, xs3_ref):
    dinv = dinv_ref[...]
    p = jnp.concatenate([palo_ref[...] + pblo_ref[...],
                         pahi_ref[...] + pbhi_ref[...]], axis=1)
    s = dinv * p + invdeg_ref[...] * xw2_ref[...] + b2_ref[...]
    h = _in_relu(s) + xwr_ref[...]
    hw3 = jnp.dot(h, w3_ref[...], preferred_element_type=jnp.float32)
    h_ref[...] = h
    hw3_ref[...] = hw3
    xs3_ref[...] = dinv * hw3


def _t3_body(pa_ref, pb_ref, hw3_ref, dinv_ref, invdeg_ref, b3_ref, out_ref):
    s = (dinv_ref[...] * (pa_ref[...] + pb_ref[...])
         + invdeg_ref[...] * hw3_ref[...] + b3_ref[...])
    out_ref[...] = jax.nn.sigmoid(s) - 0.5


def _row_spec(d):
    return pl.BlockSpec((_RB, d), lambda i: (i, 0))


def _full_spec(r, c):
    return pl.BlockSpec((r, c), lambda i: (0, 0))


def kernel(x, edge_index, W1, b1, W2, b2, Wr, br, W3, b3):
    n, in_dim = x.shape
    hid = W1.shape[1]
    e = edge_index.shape[1]
    e_pad = -(-e // (_NW * _CHUNK)) * (_NW * _CHUNK)
    per_w = e_pad // _NW
    n_chunks = per_w // _CHUNK
    i32 = jnp.int32
    fill = jnp.full((e_pad - e,), _NPAD - 1, i32)
    src = jnp.concatenate([edge_index[0], fill]).reshape(_NW, n_chunks, _CHUNK)
    dst = jnp.concatenate([edge_index[1], fill]).reshape(_NW, n_chunks, _CHUNK)

    prop_h = _make_prop(_NPAD, e_pad, _DH, 2)
    prop_16 = _make_prop(_NPAD, e_pad, 16, 1)
    deg_k = _make_deg(_NPAD, e_pad)
    zeros_h = jnp.zeros((_ZR, _DH), jnp.float32)
    zeros_16 = jnp.zeros((_ZR, 16), jnp.float32)
    ones_rows = jnp.ones((_CHUNK, 16), jnp.float32)

    grid = (n // _RB,)
    f32 = jnp.float32

    degp = deg_k(ones_rows, dst, zeros_16)
    xw1, xwr = pl.pallas_call(
        _m1_body,
        grid=grid,
        in_specs=[_row_spec(in_dim), _full_spec(in_dim, hid),
                  _full_spec(in_dim, hid), _full_spec(1, hid)],
        out_specs=[_row_spec(hid), _row_spec(hid)],
        out_shape=[jax.ShapeDtypeStruct((n, hid), f32)] * 2,
    )(x, W1, Wr, br.reshape(1, hid))

    xs1lo, xs1hi, dinv, invdeg = pl.pallas_call(
        _s1_body,
        grid=grid,
        in_specs=[_row_spec(16), _row_spec(16), _row_spec(hid)],
        out_specs=[_row_spec(_DH), _row_spec(_DH), _row_spec(1), _row_spec(1)],
        out_shape=[jax.ShapeDtypeStruct((n, _DH), f32)] * 2
        + [jax.ShapeDtypeStruct((n, 1), f32)] * 2,
    )(degp[0], degp[1], xw1)

    p1lo, p1hi = prop_h(xs1lo, xs1hi, src, dst, zeros_h)
    xw2, xs2lo, xs2hi = pl.pallas_call(
        _t1_body,
        grid=grid,
        in_specs=[_row_spec(_DH)] * 4
        + [_row_spec(hid), _row_spec(1), _row_spec(1), _full_spec(1, hid),
           _full_spec(hid, hid)],
        out_specs=[_row_spec(hid), _row_spec(_DH), _row_spec(_DH)],
        out_shape=[jax.ShapeDtypeStruct((n, hid), f32)]
        + [jax.ShapeDtypeStruct((n, _DH), f32)] * 2,
    )(p1lo[0], p1hi[0], p1lo[1], p1hi[1], xw1, dinv, invdeg,
      b1.reshape(1, hid), W2)

    w3p = jnp.zeros((hid, 16), f32).at[:, : W3.shape[1]].set(W3)
    b3p = jnp.zeros((1, 16), f32).at[0, : W3.shape[1]].set(b3)
    p2lo, p2hi = prop_h(xs2lo, xs2hi, src, dst, zeros_h)
    h, hw3, xs3 = pl.pallas_call(
        _t2_body,
        grid=grid,
        in_specs=[_row_spec(_DH)] * 4
        + [_row_spec(hid), _row_spec(1), _row_spec(1), _full_spec(1, hid),
           _row_spec(hid), _full_spec(hid, 16)],
        out_specs=[_row_spec(hid), _row_spec(16), _row_spec(16)],
        out_shape=[jax.ShapeDtypeStruct((n, hid), f32),
                   jax.ShapeDtypeStruct((n, 16), f32),
                   jax.ShapeDtypeStruct((n, 16), f32)],
    )(p2lo[0], p2hi[0], p2lo[1], p2hi[1], xw2, dinv, invdeg,
      b2.reshape(1, hid), xwr, w3p)

    (p3,) = prop_16(xs3, src, dst, zeros_16)
    outp = pl.pallas_call(
        _t3_body,
        grid=grid,
        in_specs=[_row_spec(16), _row_spec(16), _row_spec(16),
                  _row_spec(1), _row_spec(1), _full_spec(1, 16)],
        out_specs=_row_spec(16),
        out_shape=jax.ShapeDtypeStruct((n, 16), f32),
    )(p3[0], p3[1], hw3, dinv, invdeg, b3p)

    return (h, outp[:, : W3.shape[1]])
